# baseline (speedup 1.0000x reference)
import functools

import jax
import jax.numpy as jnp
from jax import lax
from jax.experimental import pallas as pl
from jax.experimental.pallas import tpu as pltpu
from jax.experimental.pallas import tpu_sc as plsc

N = 10000
E = 320000
D = 128
H = 8
DH = 16

NC = 2
NS = 16
NW = NC * NS
EPW = E // NW
CH = 128
NCH = EPW // CH
TCH = EPW - NCH * CH
NP = 10112
NPS = NP // NS

EH = E // 2
RB = 3200


def _vmesh():
    return plsc.VectorSubcoreMesh(core_axis_name="c", subcore_axis_name="s")


def _sc_gather_kvq(src, dst, KV, Q, e_cnt):
    epw = e_cnt // NW
    nch = epw // CH
    tch = epw - nch * CH
    @functools.partial(
        pl.kernel,
        mesh=_vmesh(),
        out_type=[jax.ShapeDtypeStruct((e_cnt, D), jnp.float32)] * 2,
        scratch_types=[pltpu.VMEM((CH,), jnp.int32),
                       pltpu.VMEM((CH,), jnp.int32),
                       pltpu.VMEM((CH,), jnp.int32),
                       pltpu.VMEM((CH,), jnp.int32),
                       pltpu.VMEM((CH, D), jnp.float32),
                       pltpu.VMEM((CH, D), jnp.float32),
                       pltpu.VMEM((CH, D), jnp.float32),
                       pltpu.VMEM((CH, D), jnp.float32),
                       pltpu.VMEM((tch, D), jnp.float32),
                       pltpu.VMEM((tch, D), jnp.float32),
                       pltpu.SemaphoreType.DMA,
                       pltpu.SemaphoreType.DMA],
    )
    def gk(src_hbm, dst_hbm, kv_hbm, q_hbm, kv_out, qd_out,
           sA, dA, sB, dB, kvA, qA, kvB, qB,
           kvT, qT, semA, semB):
        wid = lax.axis_index("c") * NS + lax.axis_index("s")
        base = wid * epw

        def run(off, n, si, di, kvb, qb, sem):
            pltpu.sync_copy(src_hbm.at[pl.ds(off, n)], si.at[pl.ds(0, n)])
            pltpu.sync_copy(dst_hbm.at[pl.ds(off, n)], di.at[pl.ds(0, n)])
            ckv = pltpu.async_copy(kv_hbm.at[si.at[pl.ds(0, n)]], kvb, sem)
            cq = pltpu.async_copy(q_hbm.at[di.at[pl.ds(0, n)]], qb, sem)
            return (ckv, cq)

        def fin(off, n, kvb, qb, cps):
            for cp in cps:
                cp.wait()
            pltpu.sync_copy(kvb, kv_out.at[pl.ds(off, n)])
            pltpu.sync_copy(qb, qd_out.at[pl.ds(off, n)])

        @pl.loop(0, nch // 2)
        def _(j):
            offA = base + (2 * j) * CH
            offB = offA + CH
            cA = run(offA, CH, sA, dA, kvA, qA, semA)
            cB = run(offB, CH, sB, dB, kvB, qB, semB)
            fin(offA, CH, kvA, qA, cA)
            fin(offB, CH, kvB, qB, cB)

        if nch % 2:
            offO = base + (nch - 1) * CH
            cO = run(offO, CH, sA, dA, kvA, qA, semA)
            fin(offO, CH, kvA, qA, cO)
        if tch:
            offT = base + nch * CH
            cT = run(offT, tch, sB, dB, kvT, qT, semB)
            fin(offT, tch, kvT, qT, cT)

    return gk(src, dst, KV, Q)


def _sc_scatter(dst, rows, init, e_cnt):
    epw = e_cnt // NW
    nch = epw // CH
    tch = epw - nch * CH
    @functools.partial(
        pl.kernel,
        mesh=_vmesh(),
        out_type=jax.ShapeDtypeStruct((NC, NP, D), jnp.float32),
        scratch_types=[pltpu.VMEM_SHARED((NP, D), jnp.float32),
                       pltpu.VMEM((1, CH), jnp.int32),
                       pltpu.VMEM((1, CH), jnp.int32),
                       pltpu.VMEM((CH, D), jnp.float32),
                       pltpu.VMEM((CH, D), jnp.float32),
                       pltpu.VMEM((1, tch), jnp.int32),
                       pltpu.VMEM((tch, D), jnp.float32),
                       pltpu.SemaphoreType.DMA,
                       pltpu.SemaphoreType.DMA,
                       pltpu.SemaphoreType.DMA,
                       pltpu.SemaphoreType.DMA],
    )
    def sk(dst_hbm, rows_hbm, init_hbm, out_hbm, acc, iA, iB, bA, bB, iT, bT,
           dsA, dsB, ssA, ssB):
        c = lax.axis_index("c")
        s = lax.axis_index("s")
        wid = c * NS + s
        base = wid * epw
        nsl = pl.ds(s * NPS, NPS)
        pltpu.sync_copy(init_hbm.at[c, nsl], acc.at[nsl])
        plsc.subcore_barrier()

        def chunk2(offA, offB):
            pltpu.sync_copy(dst_hbm.at[pl.ds(offA, CH)], iA.at[0])
            pltpu.sync_copy(dst_hbm.at[pl.ds(offB, CH)], iB.at[0])
            cA = pltpu.async_copy(rows_hbm.at[pl.ds(offA, CH)], bA, dsA)
            cB = pltpu.async_copy(rows_hbm.at[pl.ds(offB, CH)], bB, dsB)
            cA.wait()
            sctA = pltpu.async_copy(bA, acc.at[iA.at[0]], ssA, add=True)
            cB.wait()
            sctB = pltpu.async_copy(bB, acc.at[iB.at[0]], ssB, add=True)
            sctA.wait()
            sctB.wait()

        @pl.loop(0, nch // 2)
        def _(j):
            offA = base + (2 * j) * CH
            chunk2(offA, offA + CH)

        if nch % 2:
            offO = base + (nch - 1) * CH
            pltpu.sync_copy(dst_hbm.at[pl.ds(offO, CH)], iA.at[0])
            pltpu.sync_copy(rows_hbm.at[pl.ds(offO, CH)], bA)
            pltpu.sync_copy(bA, acc.at[iA.at[0]], add=True)
        if tch:
            offT = base + nch * CH
            pltpu.sync_copy(dst_hbm.at[pl.ds(offT, tch)], iT.at[0])
            pltpu.sync_copy(rows_hbm.at[pl.ds(offT, tch)], bT)
            pltpu.sync_copy(bT, acc.at[iT.at[0]], add=True)

        plsc.subcore_barrier()
        pltpu.sync_copy(acc.at[nsl], out_hbm.at[c, nsl])

    return sk(dst, rows, init)


def _bf16_bits(x):
    b = lax.bitcast_convert_type(x.astype(jnp.bfloat16), jnp.uint16)
    return b.astype(jnp.uint32)


def _qkv_body(h_ref, wq_ref, wx_ref, wy_ref, q_ref, kv_ref):
    h = h_ref[...]
    q_ref[...] = jnp.dot(h, wq_ref[...], preferred_element_type=jnp.float32)
    x = jnp.dot(h, wx_ref[...], preferred_element_type=jnp.float32)
    y = jnp.dot(h, wy_ref[...], preferred_element_type=jnp.float32)
    word = _bf16_bits(x) | (_bf16_bits(y) << 16)
    kv_ref[...] = lax.bitcast_convert_type(word, jnp.float32)


def _qkv(h, WQ, Wx, Wy):
    return pl.pallas_call(
        _qkv_body,
        out_shape=[jax.ShapeDtypeStruct((N, D), jnp.float32),
                   jax.ShapeDtypeStruct((N, D), jnp.float32)],
    )(h, WQ, Wx, Wy)


def _epass1_body(e_ref, kv_ref, qd_ref, we_ref, woe_ref, boe_ref,
                 ex_ref, wex_ref, z_ref, st_ref):
    i = pl.program_id(0)
    D2 = D // 2
    e = e_ref[...]
    pe = jnp.dot(e.astype(jnp.bfloat16), we_ref[...].astype(jnp.bfloat16),
                 preferred_element_type=jnp.float32)
    w = lax.bitcast_convert_type(kv_ref[...], jnp.uint32)
    lo = lax.bitcast_convert_type(w << 16, jnp.float32)
    hi = lax.bitcast_convert_type(w & jnp.uint32(0xFFFF0000), jnp.float32)
    qd = qd_ref[...]
    prod_lo = (lo[:, :D2] * qd[:, :D2]).astype(jnp.bfloat16)
    prod_hi = (hi[:, :D2] * qd[:, D2:]).astype(jnp.bfloat16)
    r2 = lax.broadcasted_iota(jnp.int32, (D2, D), 0) // DH
    cc = lax.broadcasted_iota(jnp.int32, (D2, D), 1) // DH
    Mlo = jnp.where(r2 == cc, 1.0, 0.0).astype(jnp.bfloat16)
    Mhi = jnp.where(r2 + (D2 // DH) == cc, 1.0, 0.0).astype(jnp.bfloat16)
    s0 = (jnp.dot(prod_lo, Mlo, preferred_element_type=jnp.float32)
          + jnp.dot(prod_hi, Mhi, preferred_element_type=jnp.float32)) * 0.25
    e_att = s0 + pe
    ex = jnp.exp(e_att)
    ex_ref[...] = ex
    wex_ref[...] = jnp.concatenate(
        [lo[:, D2:] * ex[:, :D2], hi[:, D2:] * ex[:, D2:]], axis=1)
    z = e + jnp.dot(e_att.astype(jnp.bfloat16),
                    woe_ref[...].astype(jnp.bfloat16),
                    preferred_element_type=jnp.float32) + boe_ref[...]
    z_ref[...] = z
    part = jnp.concatenate(
        [jnp.sum(z, axis=0)[None, :], jnp.sum(z * z, axis=0)[None, :],
         jnp.zeros((6, D), jnp.float32)], axis=0)

    @pl.when(i == 0)
    def _():
        st_ref[...] = part

    @pl.when(i > 0)
    def _():
        st_ref[...] = st_ref[...] + part


def _epass1(e, kvb, qdst, WE, WOe, bOe, e_cnt):
    return pl.pallas_call(
        _epass1_body,
        grid=(e_cnt // RB,),
        in_specs=[pl.BlockSpec((RB, D), lambda i: (i, 0)),
                  pl.BlockSpec((RB, D), lambda i: (i, 0)),
                  pl.BlockSpec((RB, D), lambda i: (i, 0)),
                  pl.BlockSpec((D, D), lambda i: (0, 0)),
                  pl.BlockSpec((D, D), lambda i: (0, 0)),
                  pl.BlockSpec((1, D), lambda i: (0, 0))],
        out_specs=[pl.BlockSpec((RB, D), lambda i: (i, 0)),
                   pl.BlockSpec((RB, D), lambda i: (i, 0)),
                   pl.BlockSpec((RB, D), lambda i: (i, 0)),
                   pl.BlockSpec((8, D), lambda i: (0, 0))],
        out_shape=[jax.ShapeDtypeStruct((e_cnt, D), jnp.float32),
                   jax.ShapeDtypeStruct((e_cnt, D), jnp.float32),
                   jax.ShapeDtypeStruct((e_cnt, D), jnp.float32),
                   jax.ShapeDtypeStruct((8, D), jnp.float32)],
    )(e, kvb, qdst, WE, WOe, bOe.reshape(1, D))


def _epass2_body(z_ref, sta_ref, stb_ref, g1_ref, b1_ref, w1_ref, bf1_ref,
                 w2_ref, bf2_ref, f_ref, st2_ref):
    i = pl.program_id(0)
    st = sta_ref[...] + stb_ref[...]
    mu = st[0:1, :] / float(E)
    var = st[1:2, :] / float(E) - mu * mu
    inv = g1_ref[...] / jnp.sqrt(var + 1e-5)
    u = (z_ref[...] - mu) * inv + b1_ref[...]
    hid = jnp.maximum(
        jnp.dot(u.astype(jnp.bfloat16), w1_ref[...].astype(jnp.bfloat16),
                preferred_element_type=jnp.float32)
        + bf1_ref[...], 0.0)
    f = u + jnp.dot(hid.astype(jnp.bfloat16),
                    w2_ref[...].astype(jnp.bfloat16),
                    preferred_element_type=jnp.float32) + bf2_ref[...]
    f_ref[...] = f
    part = jnp.concatenate(
        [jnp.sum(f, axis=0)[None, :], jnp.sum(f * f, axis=0)[None, :],
         jnp.zeros((6, D), jnp.float32)], axis=0)

    @pl.when(i == 0)
    def _():
        st2_ref[...] = part

    @pl.when(i > 0)
    def _():
        st2_ref[...] = st2_ref[...] + part


def _epass2(z, st1a, st1b, g1e, b1e, W1e, bF1e, W2e, bF2e, e_cnt):
    return pl.pallas_call(
        _epass2_body,
        grid=(e_cnt // RB,),
        in_specs=[pl.BlockSpec((RB, D), lambda i: (i, 0)),
                  pl.BlockSpec((8, D), lambda i: (0, 0)),
                  pl.BlockSpec((8, D), lambda i: (0, 0)),
                  pl.BlockSpec((1, D), lambda i: (0, 0)),
                  pl.BlockSpec((1, D), lambda i: (0, 0)),
                  pl.BlockSpec((D, 2 * D), lambda i: (0, 0)),
                  pl.BlockSpec((1, 2 * D), lambda i: (0, 0)),
                  pl.BlockSpec((2 * D, D), lambda i: (0, 0)),
                  pl.BlockSpec((1, D), lambda i: (0, 0))],
        out_specs=[pl.BlockSpec((RB, D), lambda i: (i, 0)),
                   pl.BlockSpec((8, D), lambda i: (0, 0))],
        out_shape=[jax.ShapeDtypeStruct((e_cnt, D), jnp.float32),
                   jax.ShapeDtypeStruct((8, D), jnp.float32)],
    )(z, st1a, st1b, g1e.reshape(1, D), b1e.reshape(1, D), W1e,
      bF1e.reshape(1, 2 * D), W2e, bF2e.reshape(1, D))


def _epass3_body(f1_ref, f2_ref, sta_ref, stb_ref, g2_ref, b2_ref, o_ref):
    i = pl.program_id(0)
    half = EH // RB
    st = sta_ref[...] + stb_ref[...]
    mu = st[0:1, :] / float(E)
    var = st[1:2, :] / float(E) - mu * mu
    inv = g2_ref[...] / jnp.sqrt(var + 1e-5)

    @pl.when(i < half)
    def _():
        o_ref[...] = (f1_ref[...] - mu) * inv + b2_ref[...]

    @pl.when(i >= half)
    def _():
        o_ref[...] = (f2_ref[...] - mu) * inv + b2_ref[...]


def _epass3(f1, f2, st2a, st2b, g2e, b2e):
    half = EH // RB
    return pl.pallas_call(
        _epass3_body,
        grid=(E // RB,),
        in_specs=[pl.BlockSpec((RB, D),
                               lambda i: (jnp.minimum(i, half - 1), 0)),
                  pl.BlockSpec((RB, D),
                               lambda i: (jnp.maximum(i - half, 0), 0)),
                  pl.BlockSpec((8, D), lambda i: (0, 0)),
                  pl.BlockSpec((8, D), lambda i: (0, 0)),
                  pl.BlockSpec((1, D), lambda i: (0, 0)),
                  pl.BlockSpec((1, D), lambda i: (0, 0))],
        out_specs=pl.BlockSpec((RB, D), lambda i: (i, 0)),
        out_shape=jax.ShapeDtypeStruct((E, D), jnp.float32),
    )(f1, f2, st2a, st2b, g2e.reshape(1, D), b2e.reshape(1, D))


def _hside_body(h_ref, num_ref, den_ref, woh_ref, boh_ref, g1_ref, b1_ref,
                w1_ref, bf1_ref, w2_ref, bf2_ref, g2_ref, b2_ref, o_ref):
    num = num_ref[0] + num_ref[1]
    den = den_ref[0] + den_ref[1]
    wv = jnp.where(den > 0.0, num / den, 0.0)
    h2 = h_ref[...] + jnp.dot(wv, woh_ref[...],
                              preferred_element_type=jnp.float32) + boh_ref[...]
    mu = jnp.mean(h2, axis=0, keepdims=True)
    var = jnp.mean((h2 - mu) * (h2 - mu), axis=0, keepdims=True)
    h2 = (h2 - mu) / jnp.sqrt(var + 1e-5) * g1_ref[...] + b1_ref[...]
    hid = jnp.maximum(
        jnp.dot(h2, w1_ref[...], preferred_element_type=jnp.float32)
        + bf1_ref[...], 0.0)
    h3 = h2 + jnp.dot(hid, w2_ref[...],
                      preferred_element_type=jnp.float32) + bf2_ref[...]
    mu2 = jnp.mean(h3, axis=0, keepdims=True)
    var2 = jnp.mean((h3 - mu2) * (h3 - mu2), axis=0, keepdims=True)
    o_ref[...] = (h3 - mu2) / jnp.sqrt(var2 + 1e-5) * g2_ref[...] + b2_ref[...]


def _hside(h, num_p, den_p, WOh, bOh, g1h, b1h, W1h, bF1h, W2h, bF2h,
           g2h, b2h):
    return pl.pallas_call(
        _hside_body,
        out_shape=jax.ShapeDtypeStruct((N, D), jnp.float32),
    )(h, num_p, den_p, WOh, bOh.reshape(1, D), g1h.reshape(1, D),
      b1h.reshape(1, D), W1h, bF1h.reshape(1, 2 * D), W2h,
      bF2h.reshape(1, D), g2h.reshape(1, D), b2h.reshape(1, D))


def kernel(h, e, edge_index, WQ, WK, WV, WE, WOh, bOh, WOe, bOe,
           g1h, b1h, g1e, b1e, W1h, bF1h, W2h, bF2h,
           W1e, bF1e, W2e, bF2e, g2h, b2h, g2e, b2e):
    src = edge_index[0]
    dst = edge_index[1]
    D2 = D // 2
    Wx = jnp.concatenate([WK[:, :D2], WV[:, :D2]], axis=1)
    Wy = jnp.concatenate([WK[:, D2:], WV[:, D2:]], axis=1)
    Q, KVp = _qkv(h, WQ, Wx, Wy)

    src1, src2 = src[:EH], src[EH:]
    dst1, dst2 = dst[:EH], dst[EH:]
    e1, e2 = e[:EH], e[EH:]

    kv1, qd1 = _sc_gather_kvq(src1, dst1, KVp, Q, EH)
    kv2, qd2 = _sc_gather_kvq(src2, dst2, KVp, Q, EH)
    ex1, wex1, z1, st1a = _epass1(e1, kv1, qd1, WE, WOe, bOe, EH)
    ex2, wex2, z2, st1b = _epass1(e2, kv2, qd2, WE, WOe, bOe, EH)

    zeros_nd = jnp.zeros((NC, NP, D), jnp.float32)
    den_p = _sc_scatter(dst2, ex2, _sc_scatter(dst1, ex1, zeros_nd, EH),
                        EH)[:, :N]
    num_p = _sc_scatter(dst2, wex2, _sc_scatter(dst1, wex1, zeros_nd, EH),
                        EH)[:, :N]
    h3 = _hside(h, num_p, den_p, WOh, bOh, g1h, b1h, W1h, bF1h, W2h, bF2h,
                g2h, b2h)
    f1, st2a = _epass2(z1, st1a, st1b, g1e, b1e, W1e, bF1e, W2e, bF2e, EH)
    f2, st2b = _epass2(z2, st1a, st1b, g1e, b1e, W1e, bF1e, W2e, bF2e, EH)
    e3 = _epass3(f1, f2, st2a, st2b, g2e, b2e)
    return h3, e3


# speedup vs baseline: 1.0735x; 1.0735x over previous
import functools

import jax
import jax.numpy as jnp
from jax import lax
from jax.experimental import pallas as pl
from jax.experimental.pallas import tpu as pltpu
from jax.experimental.pallas import tpu_sc as plsc

N = 10000
E = 320000
D = 128
H = 8
DH = 16

NC = 2
NS = 16
NW = NC * NS
EPW = E // NW
CH = 128
NCH = EPW // CH
TCH = EPW - NCH * CH
NP = 10112
NPS = NP // NS

EH = E // 2
RB = 3200


def _vmesh():
    return plsc.VectorSubcoreMesh(core_axis_name="c", subcore_axis_name="s")


def _sc_gather_kvq(src, dst, KV, Q, e_cnt, e_lo):
    epw = e_cnt // NW
    nch = epw // CH
    tch = epw - nch * CH
    @functools.partial(
        pl.kernel,
        mesh=_vmesh(),
        out_type=[jax.ShapeDtypeStruct((e_cnt, D), jnp.float32)] * 2,
        scratch_types=[pltpu.VMEM((CH,), jnp.int32),
                       pltpu.VMEM((CH,), jnp.int32),
                       pltpu.VMEM((CH,), jnp.int32),
                       pltpu.VMEM((CH,), jnp.int32),
                       pltpu.VMEM((CH, D), jnp.float32),
                       pltpu.VMEM((CH, D), jnp.float32),
                       pltpu.VMEM((CH, D), jnp.float32),
                       pltpu.VMEM((CH, D), jnp.float32),
                       pltpu.VMEM((tch, D), jnp.float32),
                       pltpu.VMEM((tch, D), jnp.float32),
                       pltpu.SemaphoreType.DMA,
                       pltpu.SemaphoreType.DMA],
    )
    def gk(src_hbm, dst_hbm, kv_hbm, q_hbm, kv_out, qd_out,
           sA, dA, sB, dB, kvA, qA, kvB, qB,
           kvT, qT, semA, semB):
        wid = lax.axis_index("c") * NS + lax.axis_index("s")
        base = wid * epw

        def run(off, n, si, di, kvb, qb, sem):
            pltpu.sync_copy(src_hbm.at[pl.ds(e_lo + off, n)],
                            si.at[pl.ds(0, n)])
            pltpu.sync_copy(dst_hbm.at[pl.ds(e_lo + off, n)],
                            di.at[pl.ds(0, n)])
            ckv = pltpu.async_copy(kv_hbm.at[si.at[pl.ds(0, n)]], kvb, sem)
            cq = pltpu.async_copy(q_hbm.at[di.at[pl.ds(0, n)]], qb, sem)
            return (ckv, cq)

        def fin(off, n, kvb, qb, cps):
            for cp in cps:
                cp.wait()
            pltpu.sync_copy(kvb, kv_out.at[pl.ds(off, n)])
            pltpu.sync_copy(qb, qd_out.at[pl.ds(off, n)])

        @pl.loop(0, nch // 2)
        def _(j):
            offA = base + (2 * j) * CH
            offB = offA + CH
            cA = run(offA, CH, sA, dA, kvA, qA, semA)
            cB = run(offB, CH, sB, dB, kvB, qB, semB)
            fin(offA, CH, kvA, qA, cA)
            fin(offB, CH, kvB, qB, cB)

        if nch % 2:
            offO = base + (nch - 1) * CH
            cO = run(offO, CH, sA, dA, kvA, qA, semA)
            fin(offO, CH, kvA, qA, cO)
        if tch:
            offT = base + nch * CH
            cT = run(offT, tch, sB, dB, kvT, qT, semB)
            fin(offT, tch, kvT, qT, cT)

    return gk(src, dst, KV, Q)


def _sc_scatter(dst, rows, init, e_cnt, e_lo):
    epw = e_cnt // NW
    nch = epw // CH
    tch = epw - nch * CH
    @functools.partial(
        pl.kernel,
        mesh=_vmesh(),
        out_type=jax.ShapeDtypeStruct((NC, NP, D), jnp.float32),
        scratch_types=[pltpu.VMEM_SHARED((NP, D), jnp.float32),
                       pltpu.VMEM((1, CH), jnp.int32),
                       pltpu.VMEM((1, CH), jnp.int32),
                       pltpu.VMEM((CH, D), jnp.float32),
                       pltpu.VMEM((CH, D), jnp.float32),
                       pltpu.VMEM((1, tch), jnp.int32),
                       pltpu.VMEM((tch, D), jnp.float32),
                       pltpu.SemaphoreType.DMA,
                       pltpu.SemaphoreType.DMA,
                       pltpu.SemaphoreType.DMA,
                       pltpu.SemaphoreType.DMA],
    )
    def sk(dst_hbm, rows_hbm, init_hbm, out_hbm, acc, iA, iB, bA, bB, iT, bT,
           dsA, dsB, ssA, ssB):
        c = lax.axis_index("c")
        s = lax.axis_index("s")
        wid = c * NS + s
        base = wid * epw
        nsl = pl.ds(s * NPS, NPS)
        pltpu.sync_copy(init_hbm.at[c, nsl], acc.at[nsl])
        plsc.subcore_barrier()

        def chunk2(offA, offB):
            pltpu.sync_copy(dst_hbm.at[pl.ds(e_lo + offA, CH)], iA.at[0])
            pltpu.sync_copy(dst_hbm.at[pl.ds(e_lo + offB, CH)], iB.at[0])
            cA = pltpu.async_copy(rows_hbm.at[pl.ds(offA, CH)], bA, dsA)
            cB = pltpu.async_copy(rows_hbm.at[pl.ds(offB, CH)], bB, dsB)
            cA.wait()
            sctA = pltpu.async_copy(bA, acc.at[iA.at[0]], ssA, add=True)
            cB.wait()
            sctB = pltpu.async_copy(bB, acc.at[iB.at[0]], ssB, add=True)
            sctA.wait()
            sctB.wait()

        @pl.loop(0, nch // 2)
        def _(j):
            offA = base + (2 * j) * CH
            chunk2(offA, offA + CH)

        if nch % 2:
            offO = base + (nch - 1) * CH
            pltpu.sync_copy(dst_hbm.at[pl.ds(e_lo + offO, CH)], iA.at[0])
            pltpu.sync_copy(rows_hbm.at[pl.ds(offO, CH)], bA)
            pltpu.sync_copy(bA, acc.at[iA.at[0]], add=True)
        if tch:
            offT = base + nch * CH
            pltpu.sync_copy(dst_hbm.at[pl.ds(e_lo + offT, tch)], iT.at[0])
            pltpu.sync_copy(rows_hbm.at[pl.ds(offT, tch)], bT)
            pltpu.sync_copy(bT, acc.at[iT.at[0]], add=True)

        plsc.subcore_barrier()
        pltpu.sync_copy(acc.at[nsl], out_hbm.at[c, nsl])

    return sk(dst, rows, init)


def _bf16_bits(x):
    b = lax.bitcast_convert_type(x.astype(jnp.bfloat16), jnp.uint16)
    return b.astype(jnp.uint32)


def _qkv_body(h_ref, wq_ref, wx_ref, wy_ref, q_ref, kv_ref):
    h = h_ref[...]
    q_ref[...] = jnp.dot(h, wq_ref[...], preferred_element_type=jnp.float32)
    x = jnp.dot(h, wx_ref[...], preferred_element_type=jnp.float32)
    y = jnp.dot(h, wy_ref[...], preferred_element_type=jnp.float32)
    word = _bf16_bits(x) | (_bf16_bits(y) << 16)
    kv_ref[...] = lax.bitcast_convert_type(word, jnp.float32)


def _qkv(h, WQ, Wx, Wy):
    return pl.pallas_call(
        _qkv_body,
        out_shape=[jax.ShapeDtypeStruct((N, D), jnp.float32),
                   jax.ShapeDtypeStruct((N, D), jnp.float32)],
    )(h, WQ, Wx, Wy)


def _epass1_body(e_ref, kv_ref, qd_ref, we_ref, woe_ref, boe_ref,
                 ex_ref, wex_ref, z_ref, st_ref):
    i = pl.program_id(0)
    D2 = D // 2
    e = e_ref[...]
    pe = jnp.dot(e.astype(jnp.bfloat16), we_ref[...].astype(jnp.bfloat16),
                 preferred_element_type=jnp.float32)
    w = lax.bitcast_convert_type(kv_ref[...], jnp.uint32)
    lo = lax.bitcast_convert_type(w << 16, jnp.float32)
    hi = lax.bitcast_convert_type(w & jnp.uint32(0xFFFF0000), jnp.float32)
    qd = qd_ref[...]
    prod_lo = (lo[:, :D2] * qd[:, :D2]).astype(jnp.bfloat16)
    prod_hi = (hi[:, :D2] * qd[:, D2:]).astype(jnp.bfloat16)
    r2 = lax.broadcasted_iota(jnp.int32, (D2, D), 0) // DH
    cc = lax.broadcasted_iota(jnp.int32, (D2, D), 1) // DH
    Mlo = jnp.where(r2 == cc, 1.0, 0.0).astype(jnp.bfloat16)
    Mhi = jnp.where(r2 + (D2 // DH) == cc, 1.0, 0.0).astype(jnp.bfloat16)
    s0 = (jnp.dot(prod_lo, Mlo, preferred_element_type=jnp.float32)
          + jnp.dot(prod_hi, Mhi, preferred_element_type=jnp.float32)) * 0.25
    e_att = s0 + pe
    ex = jnp.exp(e_att)
    ex_ref[...] = ex
    wex_ref[...] = jnp.concatenate(
        [lo[:, D2:] * ex[:, :D2], hi[:, D2:] * ex[:, D2:]], axis=1)
    z = e + jnp.dot(e_att.astype(jnp.bfloat16),
                    woe_ref[...].astype(jnp.bfloat16),
                    preferred_element_type=jnp.float32) + boe_ref[...]
    z_ref[...] = z
    part = jnp.concatenate(
        [jnp.sum(z, axis=0)[None, :], jnp.sum(z * z, axis=0)[None, :],
         jnp.zeros((6, D), jnp.float32)], axis=0)

    @pl.when(i == 0)
    def _():
        st_ref[...] = part

    @pl.when(i > 0)
    def _():
        st_ref[...] = st_ref[...] + part


def _epass1(e, kvb, qdst, WE, WOe, bOe, e_cnt, blk_off):
    return pl.pallas_call(
        _epass1_body,
        grid=(e_cnt // RB,),
        in_specs=[pl.BlockSpec((RB, D), lambda i: (i + blk_off, 0)),
                  pl.BlockSpec((RB, D), lambda i: (i, 0)),
                  pl.BlockSpec((RB, D), lambda i: (i, 0)),
                  pl.BlockSpec((D, D), lambda i: (0, 0)),
                  pl.BlockSpec((D, D), lambda i: (0, 0)),
                  pl.BlockSpec((1, D), lambda i: (0, 0))],
        out_specs=[pl.BlockSpec((RB, D), lambda i: (i, 0)),
                   pl.BlockSpec((RB, D), lambda i: (i, 0)),
                   pl.BlockSpec((RB, D), lambda i: (i, 0)),
                   pl.BlockSpec((8, D), lambda i: (0, 0))],
        out_shape=[jax.ShapeDtypeStruct((e_cnt, D), jnp.float32),
                   jax.ShapeDtypeStruct((e_cnt, D), jnp.float32),
                   jax.ShapeDtypeStruct((e_cnt, D), jnp.float32),
                   jax.ShapeDtypeStruct((8, D), jnp.float32)],
    )(e, kvb, qdst, WE, WOe, bOe.reshape(1, D))


def _epass2_body(z_ref, sta_ref, stb_ref, g1_ref, b1_ref, w1_ref, bf1_ref,
                 w2_ref, bf2_ref, f_ref, st2_ref):
    i = pl.program_id(0)
    st = sta_ref[...] + stb_ref[...]
    mu = st[0:1, :] / float(E)
    var = st[1:2, :] / float(E) - mu * mu
    inv = g1_ref[...] / jnp.sqrt(var + 1e-5)
    u = (z_ref[...] - mu) * inv + b1_ref[...]
    hid = jnp.maximum(
        jnp.dot(u.astype(jnp.bfloat16), w1_ref[...].astype(jnp.bfloat16),
                preferred_element_type=jnp.float32)
        + bf1_ref[...], 0.0)
    f = u + jnp.dot(hid.astype(jnp.bfloat16),
                    w2_ref[...].astype(jnp.bfloat16),
                    preferred_element_type=jnp.float32) + bf2_ref[...]
    f_ref[...] = f
    part = jnp.concatenate(
        [jnp.sum(f, axis=0)[None, :], jnp.sum(f * f, axis=0)[None, :],
         jnp.zeros((6, D), jnp.float32)], axis=0)

    @pl.when(i == 0)
    def _():
        st2_ref[...] = part

    @pl.when(i > 0)
    def _():
        st2_ref[...] = st2_ref[...] + part


def _epass2(z, st1a, st1b, g1e, b1e, W1e, bF1e, W2e, bF2e, e_cnt):
    return pl.pallas_call(
        _epass2_body,
        grid=(e_cnt // RB,),
        in_specs=[pl.BlockSpec((RB, D), lambda i: (i, 0)),
                  pl.BlockSpec((8, D), lambda i: (0, 0)),
                  pl.BlockSpec((8, D), lambda i: (0, 0)),
                  pl.BlockSpec((1, D), lambda i: (0, 0)),
                  pl.BlockSpec((1, D), lambda i: (0, 0)),
                  pl.BlockSpec((D, 2 * D), lambda i: (0, 0)),
                  pl.BlockSpec((1, 2 * D), lambda i: (0, 0)),
                  pl.BlockSpec((2 * D, D), lambda i: (0, 0)),
                  pl.BlockSpec((1, D), lambda i: (0, 0))],
        out_specs=[pl.BlockSpec((RB, D), lambda i: (i, 0)),
                   pl.BlockSpec((8, D), lambda i: (0, 0))],
        out_shape=[jax.ShapeDtypeStruct((e_cnt, D), jnp.float32),
                   jax.ShapeDtypeStruct((8, D), jnp.float32)],
    )(z, st1a, st1b, g1e.reshape(1, D), b1e.reshape(1, D), W1e,
      bF1e.reshape(1, 2 * D), W2e, bF2e.reshape(1, D))


def _epass3_body(f1_ref, f2_ref, sta_ref, stb_ref, g2_ref, b2_ref, o_ref):
    i = pl.program_id(0)
    half = EH // RB
    st = sta_ref[...] + stb_ref[...]
    mu = st[0:1, :] / float(E)
    var = st[1:2, :] / float(E) - mu * mu
    inv = g2_ref[...] / jnp.sqrt(var + 1e-5)

    @pl.when(i < half)
    def _():
        o_ref[...] = (f1_ref[...] - mu) * inv + b2_ref[...]

    @pl.when(i >= half)
    def _():
        o_ref[...] = (f2_ref[...] - mu) * inv + b2_ref[...]


def _epass3(f1, f2, st2a, st2b, g2e, b2e):
    half = EH // RB
    return pl.pallas_call(
        _epass3_body,
        grid=(E // RB,),
        in_specs=[pl.BlockSpec((RB, D),
                               lambda i: (jnp.minimum(i, half - 1), 0)),
                  pl.BlockSpec((RB, D),
                               lambda i: (jnp.maximum(i - half, 0), 0)),
                  pl.BlockSpec((8, D), lambda i: (0, 0)),
                  pl.BlockSpec((8, D), lambda i: (0, 0)),
                  pl.BlockSpec((1, D), lambda i: (0, 0)),
                  pl.BlockSpec((1, D), lambda i: (0, 0))],
        out_specs=pl.BlockSpec((RB, D), lambda i: (i, 0)),
        out_shape=jax.ShapeDtypeStruct((E, D), jnp.float32),
    )(f1, f2, st2a, st2b, g2e.reshape(1, D), b2e.reshape(1, D))


def _hside_body(h_ref, num_ref, den_ref, woh_ref, boh_ref, g1_ref, b1_ref,
                w1_ref, bf1_ref, w2_ref, bf2_ref, g2_ref, b2_ref, o_ref):
    num = num_ref[0] + num_ref[1]
    den = den_ref[0] + den_ref[1]
    wv = jnp.where(den > 0.0, num / den, 0.0)
    h2 = h_ref[...] + jnp.dot(wv, woh_ref[...],
                              preferred_element_type=jnp.float32) + boh_ref[...]
    mu = jnp.mean(h2, axis=0, keepdims=True)
    var = jnp.mean((h2 - mu) * (h2 - mu), axis=0, keepdims=True)
    h2 = (h2 - mu) / jnp.sqrt(var + 1e-5) * g1_ref[...] + b1_ref[...]
    hid = jnp.maximum(
        jnp.dot(h2, w1_ref[...], preferred_element_type=jnp.float32)
        + bf1_ref[...], 0.0)
    h3 = h2 + jnp.dot(hid, w2_ref[...],
                      preferred_element_type=jnp.float32) + bf2_ref[...]
    mu2 = jnp.mean(h3, axis=0, keepdims=True)
    var2 = jnp.mean((h3 - mu2) * (h3 - mu2), axis=0, keepdims=True)
    o_ref[...] = (h3 - mu2) / jnp.sqrt(var2 + 1e-5) * g2_ref[...] + b2_ref[...]


def _hside(h, num_p, den_p, WOh, bOh, g1h, b1h, W1h, bF1h, W2h, bF2h,
           g2h, b2h):
    return pl.pallas_call(
        _hside_body,
        out_shape=jax.ShapeDtypeStruct((N, D), jnp.float32),
    )(h, num_p, den_p, WOh, bOh.reshape(1, D), g1h.reshape(1, D),
      b1h.reshape(1, D), W1h, bF1h.reshape(1, 2 * D), W2h,
      bF2h.reshape(1, D), g2h.reshape(1, D), b2h.reshape(1, D))


def kernel(h, e, edge_index, WQ, WK, WV, WE, WOh, bOh, WOe, bOe,
           g1h, b1h, g1e, b1e, W1h, bF1h, W2h, bF2h,
           W1e, bF1e, W2e, bF2e, g2h, b2h, g2e, b2e):
    src = edge_index[0]
    dst = edge_index[1]
    D2 = D // 2
    Wx = jnp.concatenate([WK[:, :D2], WV[:, :D2]], axis=1)
    Wy = jnp.concatenate([WK[:, D2:], WV[:, D2:]], axis=1)
    Q, KVp = _qkv(h, WQ, Wx, Wy)

    kv1, qd1 = _sc_gather_kvq(src, dst, KVp, Q, EH, 0)
    kv2, qd2 = _sc_gather_kvq(src, dst, KVp, Q, EH, EH)
    ex1, wex1, z1, st1a = _epass1(e, kv1, qd1, WE, WOe, bOe, EH, 0)
    ex2, wex2, z2, st1b = _epass1(e, kv2, qd2, WE, WOe, bOe, EH, EH // RB)

    zeros_nd = jnp.zeros((NC, NP, D), jnp.float32)
    den_p = _sc_scatter(dst, ex2,
                        _sc_scatter(dst, ex1, zeros_nd, EH, 0),
                        EH, EH)[:, :N]
    num_p = _sc_scatter(dst, wex2,
                        _sc_scatter(dst, wex1, zeros_nd, EH, 0),
                        EH, EH)[:, :N]
    h3 = _hside(h, num_p, den_p, WOh, bOh, g1h, b1h, W1h, bF1h, W2h, bF2h,
                g2h, b2h)
    f1, st2a = _epass2(z1, st1a, st1b, g1e, b1e, W1e, bF1e, W2e, bF2e, EH)
    f2, st2b = _epass2(z2, st1a, st1b, g1e, b1e, W1e, bF1e, W2e, bF2e, EH)
    e3 = _epass3(f1, f2, st2a, st2b, g2e, b2e)
    return h3, e3


# speedup vs baseline: 1.1468x; 1.0683x over previous
import functools

import jax
import jax.numpy as jnp
from jax import lax
from jax.experimental import pallas as pl
from jax.experimental.pallas import tpu as pltpu
from jax.experimental.pallas import tpu_sc as plsc

N = 10000
E = 320000
D = 128
H = 8
DH = 16

NC = 2
NS = 16
NW = NC * NS
EPW = E // NW
CH = 128
NCH = EPW // CH
TCH = EPW - NCH * CH
NP = 10112
NPS = NP // NS

EH = E // 2
RB = 3200


def _vmesh():
    return plsc.VectorSubcoreMesh(core_axis_name="c", subcore_axis_name="s")


def _sc_gather_kvq(src, dst, KV, Q, e_cnt, e_lo):
    epw = e_cnt // NW
    nch = epw // CH
    tch = epw - nch * CH
    @functools.partial(
        pl.kernel,
        mesh=_vmesh(),
        out_type=[jax.ShapeDtypeStruct((e_cnt, D), jnp.float32)] * 2,
        scratch_types=[pltpu.VMEM((CH,), jnp.int32),
                       pltpu.VMEM((CH,), jnp.int32),
                       pltpu.VMEM((CH,), jnp.int32),
                       pltpu.VMEM((CH,), jnp.int32),
                       pltpu.VMEM((CH, D), jnp.float32),
                       pltpu.VMEM((CH, D), jnp.float32),
                       pltpu.VMEM((CH, D), jnp.float32),
                       pltpu.VMEM((CH, D), jnp.float32),
                       pltpu.VMEM((tch, D), jnp.float32),
                       pltpu.VMEM((tch, D), jnp.float32),
                       pltpu.SemaphoreType.DMA,
                       pltpu.SemaphoreType.DMA],
    )
    def gk(src_hbm, dst_hbm, kv_hbm, q_hbm, kv_out, qd_out,
           sA, dA, sB, dB, kvA, qA, kvB, qB,
           kvT, qT, semA, semB):
        wid = lax.axis_index("c") * NS + lax.axis_index("s")
        base = wid * epw

        def run(off, n, si, di, kvb, qb, sem):
            pltpu.sync_copy(src_hbm.at[pl.ds(e_lo + off, n)],
                            si.at[pl.ds(0, n)])
            pltpu.sync_copy(dst_hbm.at[pl.ds(e_lo + off, n)],
                            di.at[pl.ds(0, n)])
            ckv = pltpu.async_copy(kv_hbm.at[si.at[pl.ds(0, n)]], kvb, sem)
            cq = pltpu.async_copy(q_hbm.at[di.at[pl.ds(0, n)]], qb, sem)
            return (ckv, cq)

        def fin(off, n, kvb, qb, cps):
            for cp in cps:
                cp.wait()
            pltpu.sync_copy(kvb, kv_out.at[pl.ds(off, n)])
            pltpu.sync_copy(qb, qd_out.at[pl.ds(off, n)])

        @pl.loop(0, nch // 2)
        def _(j):
            offA = base + (2 * j) * CH
            offB = offA + CH
            cA = run(offA, CH, sA, dA, kvA, qA, semA)
            cB = run(offB, CH, sB, dB, kvB, qB, semB)
            fin(offA, CH, kvA, qA, cA)
            fin(offB, CH, kvB, qB, cB)

        if nch % 2:
            offO = base + (nch - 1) * CH
            cO = run(offO, CH, sA, dA, kvA, qA, semA)
            fin(offO, CH, kvA, qA, cO)
        if tch:
            offT = base + nch * CH
            cT = run(offT, tch, sB, dB, kvT, qT, semB)
            fin(offT, tch, kvT, qT, cT)

    return gk(src, dst, KV, Q)


def _sc_scatter(dst, rows, init, e_cnt, e_lo):
    epw = e_cnt // NW
    nch = epw // CH
    tch = epw - nch * CH
    @functools.partial(
        pl.kernel,
        mesh=_vmesh(),
        out_type=jax.ShapeDtypeStruct((NC, NP, D), jnp.float32),
        scratch_types=[pltpu.VMEM_SHARED((NP, D), jnp.float32),
                       pltpu.VMEM((1, CH), jnp.int32),
                       pltpu.VMEM((1, CH), jnp.int32),
                       pltpu.VMEM((CH, D), jnp.float32),
                       pltpu.VMEM((CH, D), jnp.float32),
                       pltpu.VMEM((1, tch), jnp.int32),
                       pltpu.VMEM((tch, D), jnp.float32),
                       pltpu.SemaphoreType.DMA,
                       pltpu.SemaphoreType.DMA,
                       pltpu.SemaphoreType.DMA,
                       pltpu.SemaphoreType.DMA],
    )
    def sk(dst_hbm, rows_hbm, init_hbm, out_hbm, acc, iA, iB, bA, bB, iT, bT,
           dsA, dsB, ssA, ssB):
        c = lax.axis_index("c")
        s = lax.axis_index("s")
        wid = c * NS + s
        base = wid * epw
        nsl = pl.ds(s * NPS, NPS)
        pltpu.sync_copy(init_hbm.at[c, nsl], acc.at[nsl])
        plsc.subcore_barrier()

        def chunk2(offA, offB):
            pltpu.sync_copy(dst_hbm.at[pl.ds(e_lo + offA, CH)], iA.at[0])
            pltpu.sync_copy(dst_hbm.at[pl.ds(e_lo + offB, CH)], iB.at[0])
            cA = pltpu.async_copy(rows_hbm.at[pl.ds(offA, CH)], bA, dsA)
            cB = pltpu.async_copy(rows_hbm.at[pl.ds(offB, CH)], bB, dsB)
            cA.wait()
            sctA = pltpu.async_copy(bA, acc.at[iA.at[0]], ssA, add=True)
            cB.wait()
            sctB = pltpu.async_copy(bB, acc.at[iB.at[0]], ssB, add=True)
            sctA.wait()
            sctB.wait()

        @pl.loop(0, nch // 2)
        def _(j):
            offA = base + (2 * j) * CH
            chunk2(offA, offA + CH)

        if nch % 2:
            offO = base + (nch - 1) * CH
            pltpu.sync_copy(dst_hbm.at[pl.ds(e_lo + offO, CH)], iA.at[0])
            pltpu.sync_copy(rows_hbm.at[pl.ds(offO, CH)], bA)
            pltpu.sync_copy(bA, acc.at[iA.at[0]], add=True)
        if tch:
            offT = base + nch * CH
            pltpu.sync_copy(dst_hbm.at[pl.ds(e_lo + offT, tch)], iT.at[0])
            pltpu.sync_copy(rows_hbm.at[pl.ds(offT, tch)], bT)
            pltpu.sync_copy(bT, acc.at[iT.at[0]], add=True)

        plsc.subcore_barrier()
        pltpu.sync_copy(acc.at[nsl], out_hbm.at[c, nsl])

    return sk(dst, rows, init)


def _bf16_bits(x):
    b = lax.bitcast_convert_type(x.astype(jnp.bfloat16), jnp.uint16)
    return b.astype(jnp.uint32)


def _qkv_body(h_ref, wq_ref, wk_ref, wv_ref, q_ref, kv_ref):
    D2 = D // 2
    h = h_ref[...]
    q_ref[...] = jnp.dot(h, wq_ref[...], preferred_element_type=jnp.float32)
    k = jnp.dot(h, wk_ref[...], preferred_element_type=jnp.float32)
    v = jnp.dot(h, wv_ref[...], preferred_element_type=jnp.float32)
    x = jnp.concatenate([k[:, :D2], v[:, :D2]], axis=1)
    y = jnp.concatenate([k[:, D2:], v[:, D2:]], axis=1)
    word = _bf16_bits(x) | (_bf16_bits(y) << 16)
    kv_ref[...] = lax.bitcast_convert_type(word, jnp.float32)


def _qkv(h, WQ, WK, WV):
    return pl.pallas_call(
        _qkv_body,
        out_shape=[jax.ShapeDtypeStruct((N, D), jnp.float32),
                   jax.ShapeDtypeStruct((N, D), jnp.float32)],
    )(h, WQ, WK, WV)


def _epass1_body(e_ref, kv_ref, qd_ref, we_ref, woe_ref, boe_ref,
                 ex_ref, wex_ref, z_ref, st_ref):
    i = pl.program_id(0)
    D2 = D // 2
    e = e_ref[...]
    pe = jnp.dot(e.astype(jnp.bfloat16), we_ref[...].astype(jnp.bfloat16),
                 preferred_element_type=jnp.float32)
    w = lax.bitcast_convert_type(kv_ref[...], jnp.uint32)
    lo = lax.bitcast_convert_type(w << 16, jnp.float32)
    hi = lax.bitcast_convert_type(w & jnp.uint32(0xFFFF0000), jnp.float32)
    qd = qd_ref[...]
    prod_lo = (lo[:, :D2] * qd[:, :D2]).astype(jnp.bfloat16)
    prod_hi = (hi[:, :D2] * qd[:, D2:]).astype(jnp.bfloat16)
    r2 = lax.broadcasted_iota(jnp.int32, (D2, D), 0) // DH
    cc = lax.broadcasted_iota(jnp.int32, (D2, D), 1) // DH
    Mlo = jnp.where(r2 == cc, 1.0, 0.0).astype(jnp.bfloat16)
    Mhi = jnp.where(r2 + (D2 // DH) == cc, 1.0, 0.0).astype(jnp.bfloat16)
    s0 = (jnp.dot(prod_lo, Mlo, preferred_element_type=jnp.float32)
          + jnp.dot(prod_hi, Mhi, preferred_element_type=jnp.float32)) * 0.25
    e_att = s0 + pe
    ex = jnp.exp(e_att)
    ex_ref[...] = ex
    wex_ref[...] = jnp.concatenate(
        [lo[:, D2:] * ex[:, :D2], hi[:, D2:] * ex[:, D2:]], axis=1)
    z = e + jnp.dot(e_att.astype(jnp.bfloat16),
                    woe_ref[...].astype(jnp.bfloat16),
                    preferred_element_type=jnp.float32) + boe_ref[...]
    z_ref[...] = z.astype(jnp.bfloat16)
    part = jnp.concatenate(
        [jnp.sum(z, axis=0)[None, :], jnp.sum(z * z, axis=0)[None, :],
         jnp.zeros((6, D), jnp.float32)], axis=0)

    @pl.when(i == 0)
    def _():
        st_ref[...] = part

    @pl.when(i > 0)
    def _():
        st_ref[...] = st_ref[...] + part


def _epass1(e, kvb, qdst, WE, WOe, bOe, e_cnt, blk_off):
    return pl.pallas_call(
        _epass1_body,
        grid=(e_cnt // RB,),
        in_specs=[pl.BlockSpec((RB, D), lambda i: (i + blk_off, 0)),
                  pl.BlockSpec((RB, D), lambda i: (i, 0)),
                  pl.BlockSpec((RB, D), lambda i: (i, 0)),
                  pl.BlockSpec((D, D), lambda i: (0, 0)),
                  pl.BlockSpec((D, D), lambda i: (0, 0)),
                  pl.BlockSpec((1, D), lambda i: (0, 0))],
        out_specs=[pl.BlockSpec((RB, D), lambda i: (i, 0)),
                   pl.BlockSpec((RB, D), lambda i: (i, 0)),
                   pl.BlockSpec((RB, D), lambda i: (i, 0)),
                   pl.BlockSpec((8, D), lambda i: (0, 0))],
        out_shape=[jax.ShapeDtypeStruct((e_cnt, D), jnp.float32),
                   jax.ShapeDtypeStruct((e_cnt, D), jnp.float32),
                   jax.ShapeDtypeStruct((e_cnt, D), jnp.bfloat16),
                   jax.ShapeDtypeStruct((8, D), jnp.float32)],
    )(e, kvb, qdst, WE, WOe, bOe.reshape(1, D))


def _epass2_body(z_ref, sta_ref, stb_ref, g1_ref, b1_ref, w1_ref, bf1_ref,
                 w2_ref, bf2_ref, f_ref, st2_ref):
    i = pl.program_id(0)
    st = sta_ref[...] + stb_ref[...]
    mu = st[0:1, :] / float(E)
    var = st[1:2, :] / float(E) - mu * mu
    inv = g1_ref[...] / jnp.sqrt(var + 1e-5)
    u = (z_ref[...].astype(jnp.float32) - mu) * inv + b1_ref[...]
    hid = jnp.maximum(
        jnp.dot(u.astype(jnp.bfloat16), w1_ref[...].astype(jnp.bfloat16),
                preferred_element_type=jnp.float32)
        + bf1_ref[...], 0.0)
    f = u + jnp.dot(hid.astype(jnp.bfloat16),
                    w2_ref[...].astype(jnp.bfloat16),
                    preferred_element_type=jnp.float32) + bf2_ref[...]
    f_ref[...] = f.astype(jnp.bfloat16)
    part = jnp.concatenate(
        [jnp.sum(f, axis=0)[None, :], jnp.sum(f * f, axis=0)[None, :],
         jnp.zeros((6, D), jnp.float32)], axis=0)

    @pl.when(i == 0)
    def _():
        st2_ref[...] = part

    @pl.when(i > 0)
    def _():
        st2_ref[...] = st2_ref[...] + part


def _epass2(z, st1a, st1b, g1e, b1e, W1e, bF1e, W2e, bF2e, e_cnt):
    return pl.pallas_call(
        _epass2_body,
        grid=(e_cnt // RB,),
        in_specs=[pl.BlockSpec((RB, D), lambda i: (i, 0)),
                  pl.BlockSpec((8, D), lambda i: (0, 0)),
                  pl.BlockSpec((8, D), lambda i: (0, 0)),
                  pl.BlockSpec((1, D), lambda i: (0, 0)),
                  pl.BlockSpec((1, D), lambda i: (0, 0)),
                  pl.BlockSpec((D, 2 * D), lambda i: (0, 0)),
                  pl.BlockSpec((1, 2 * D), lambda i: (0, 0)),
                  pl.BlockSpec((2 * D, D), lambda i: (0, 0)),
                  pl.BlockSpec((1, D), lambda i: (0, 0))],
        out_specs=[pl.BlockSpec((RB, D), lambda i: (i, 0)),
                   pl.BlockSpec((8, D), lambda i: (0, 0))],
        out_shape=[jax.ShapeDtypeStruct((e_cnt, D), jnp.bfloat16),
                   jax.ShapeDtypeStruct((8, D), jnp.float32)],
    )(z, st1a, st1b, g1e.reshape(1, D), b1e.reshape(1, D), W1e,
      bF1e.reshape(1, 2 * D), W2e, bF2e.reshape(1, D))


def _epass3_body(f1_ref, f2_ref, sta_ref, stb_ref, g2_ref, b2_ref, o_ref):
    i = pl.program_id(0)
    half = EH // RB
    st = sta_ref[...] + stb_ref[...]
    mu = st[0:1, :] / float(E)
    var = st[1:2, :] / float(E) - mu * mu
    inv = g2_ref[...] / jnp.sqrt(var + 1e-5)

    @pl.when(i < half)
    def _():
        o_ref[...] = (f1_ref[...].astype(jnp.float32) - mu) * inv + b2_ref[...]

    @pl.when(i >= half)
    def _():
        o_ref[...] = (f2_ref[...].astype(jnp.float32) - mu) * inv + b2_ref[...]


def _epass3(f1, f2, st2a, st2b, g2e, b2e):
    half = EH // RB
    return pl.pallas_call(
        _epass3_body,
        grid=(E // RB,),
        in_specs=[pl.BlockSpec((RB, D),
                               lambda i: (jnp.minimum(i, half - 1), 0)),
                  pl.BlockSpec((RB, D),
                               lambda i: (jnp.maximum(i - half, 0), 0)),
                  pl.BlockSpec((8, D), lambda i: (0, 0)),
                  pl.BlockSpec((8, D), lambda i: (0, 0)),
                  pl.BlockSpec((1, D), lambda i: (0, 0)),
                  pl.BlockSpec((1, D), lambda i: (0, 0))],
        out_specs=pl.BlockSpec((RB, D), lambda i: (i, 0)),
        out_shape=jax.ShapeDtypeStruct((E, D), jnp.float32),
    )(f1, f2, st2a, st2b, g2e.reshape(1, D), b2e.reshape(1, D))


def _hside_body(h_ref, num_ref, den_ref, woh_ref, boh_ref, g1_ref, b1_ref,
                w1_ref, bf1_ref, w2_ref, bf2_ref, g2_ref, b2_ref, o_ref):
    num = num_ref[0] + num_ref[1]
    den = den_ref[0] + den_ref[1]
    wv = jnp.where(den > 0.0, num / den, 0.0)
    h2 = h_ref[...] + jnp.dot(wv, woh_ref[...],
                              preferred_element_type=jnp.float32) + boh_ref[...]
    mu = jnp.mean(h2, axis=0, keepdims=True)
    var = jnp.mean((h2 - mu) * (h2 - mu), axis=0, keepdims=True)
    h2 = (h2 - mu) / jnp.sqrt(var + 1e-5) * g1_ref[...] + b1_ref[...]
    hid = jnp.maximum(
        jnp.dot(h2, w1_ref[...], preferred_element_type=jnp.float32)
        + bf1_ref[...], 0.0)
    h3 = h2 + jnp.dot(hid, w2_ref[...],
                      preferred_element_type=jnp.float32) + bf2_ref[...]
    mu2 = jnp.mean(h3, axis=0, keepdims=True)
    var2 = jnp.mean((h3 - mu2) * (h3 - mu2), axis=0, keepdims=True)
    o_ref[...] = (h3 - mu2) / jnp.sqrt(var2 + 1e-5) * g2_ref[...] + b2_ref[...]


def _hside(h, num_p, den_p, WOh, bOh, g1h, b1h, W1h, bF1h, W2h, bF2h,
           g2h, b2h):
    return pl.pallas_call(
        _hside_body,
        out_shape=jax.ShapeDtypeStruct((N, D), jnp.float32),
    )(h, num_p, den_p, WOh, bOh.reshape(1, D), g1h.reshape(1, D),
      b1h.reshape(1, D), W1h, bF1h.reshape(1, 2 * D), W2h,
      bF2h.reshape(1, D), g2h.reshape(1, D), b2h.reshape(1, D))


def kernel(h, e, edge_index, WQ, WK, WV, WE, WOh, bOh, WOe, bOe,
           g1h, b1h, g1e, b1e, W1h, bF1h, W2h, bF2h,
           W1e, bF1e, W2e, bF2e, g2h, b2h, g2e, b2e):
    src = edge_index[0]
    dst = edge_index[1]
    Q, KVp = _qkv(h, WQ, WK, WV)

    kv1, qd1 = _sc_gather_kvq(src, dst, KVp, Q, EH, 0)
    kv2, qd2 = _sc_gather_kvq(src, dst, KVp, Q, EH, EH)
    ex1, wex1, z1, st1a = _epass1(e, kv1, qd1, WE, WOe, bOe, EH, 0)
    ex2, wex2, z2, st1b = _epass1(e, kv2, qd2, WE, WOe, bOe, EH, EH // RB)

    zeros_nd = jnp.zeros((NC, NP, D), jnp.float32)
    den_p = _sc_scatter(dst, ex2,
                        _sc_scatter(dst, ex1, zeros_nd, EH, 0),
                        EH, EH)[:, :N]
    num_p = _sc_scatter(dst, wex2,
                        _sc_scatter(dst, wex1, zeros_nd, EH, 0),
                        EH, EH)[:, :N]
    h3 = _hside(h, num_p, den_p, WOh, bOh, g1h, b1h, W1h, bF1h, W2h, bF2h,
                g2h, b2h)
    f1, st2a = _epass2(z1, st1a, st1b, g1e, b1e, W1e, bF1e, W2e, bF2e, EH)
    f2, st2b = _epass2(z2, st1a, st1b, g1e, b1e, W1e, bF1e, W2e, bF2e, EH)
    e3 = _epass3(f1, f2, st2a, st2b, g2e, b2e)
    return h3, e3


# speedup vs baseline: 1.1892x; 1.0370x over previous
import functools

import jax
import jax.numpy as jnp
from jax import lax
from jax.experimental import pallas as pl
from jax.experimental.pallas import tpu as pltpu
from jax.experimental.pallas import tpu_sc as plsc

N = 10000
E = 320000
D = 128
H = 8
DH = 16

NC = 2
NS = 16
NW = NC * NS
EPW = E // NW
CH = 128
NCH = EPW // CH
TCH = EPW - NCH * CH
NP = 10112
NPS = NP // NS

EH = E // 2
RB = 3200


def _vmesh():
    return plsc.VectorSubcoreMesh(core_axis_name="c", subcore_axis_name="s")


def _sc_gather_kvq(src, dst, KV, Q, e_cnt, e_lo):
    epw = e_cnt // NW
    nch = epw // CH
    tch = epw - nch * CH
    @functools.partial(
        pl.kernel,
        mesh=_vmesh(),
        out_type=[jax.ShapeDtypeStruct((e_cnt, D), jnp.float32)] * 2,
        scratch_types=[pltpu.VMEM((CH,), jnp.int32),
                       pltpu.VMEM((CH,), jnp.int32),
                       pltpu.VMEM((CH,), jnp.int32),
                       pltpu.VMEM((CH,), jnp.int32),
                       pltpu.VMEM((CH, D), jnp.float32),
                       pltpu.VMEM((CH, D), jnp.float32),
                       pltpu.VMEM((CH, D), jnp.float32),
                       pltpu.VMEM((CH, D), jnp.float32),
                       pltpu.VMEM((tch, D), jnp.float32),
                       pltpu.VMEM((tch, D), jnp.float32),
                       pltpu.SemaphoreType.DMA,
                       pltpu.SemaphoreType.DMA],
    )
    def gk(src_hbm, dst_hbm, kv_hbm, q_hbm, kv_out, qd_out,
           sA, dA, sB, dB, kvA, qA, kvB, qB,
           kvT, qT, semA, semB):
        wid = lax.axis_index("c") * NS + lax.axis_index("s")
        base = wid * epw

        def run(off, n, si, di, kvb, qb, sem):
            pltpu.sync_copy(src_hbm.at[pl.ds(e_lo + off, n)],
                            si.at[pl.ds(0, n)])
            pltpu.sync_copy(dst_hbm.at[pl.ds(e_lo + off, n)],
                            di.at[pl.ds(0, n)])
            ckv = pltpu.async_copy(kv_hbm.at[si.at[pl.ds(0, n)]], kvb, sem)
            cq = pltpu.async_copy(q_hbm.at[di.at[pl.ds(0, n)]], qb, sem)
            return (ckv, cq)

        def fin(off, n, kvb, qb, cps):
            for cp in cps:
                cp.wait()
            pltpu.sync_copy(kvb, kv_out.at[pl.ds(off, n)])
            pltpu.sync_copy(qb, qd_out.at[pl.ds(off, n)])

        @pl.loop(0, nch // 2)
        def _(j):
            offA = base + (2 * j) * CH
            offB = offA + CH
            cA = run(offA, CH, sA, dA, kvA, qA, semA)
            cB = run(offB, CH, sB, dB, kvB, qB, semB)
            fin(offA, CH, kvA, qA, cA)
            fin(offB, CH, kvB, qB, cB)

        if nch % 2:
            offO = base + (nch - 1) * CH
            cO = run(offO, CH, sA, dA, kvA, qA, semA)
            fin(offO, CH, kvA, qA, cO)
        if tch:
            offT = base + nch * CH
            cT = run(offT, tch, sB, dB, kvT, qT, semB)
            fin(offT, tch, kvT, qT, cT)

    return gk(src, dst, KV, Q)


def _sc_scatter2(dst, rows_den, rows_num, init, e_cnt, e_lo):
    epw = e_cnt // NS
    nch = epw // CH
    tch = epw - nch * CH
    @functools.partial(
        pl.kernel,
        mesh=_vmesh(),
        out_type=jax.ShapeDtypeStruct((NC, NP, D), jnp.float32),
        scratch_types=[pltpu.VMEM_SHARED((NP, D), jnp.float32),
                       pltpu.VMEM((1, CH), jnp.int32),
                       pltpu.VMEM((1, CH), jnp.int32),
                       pltpu.VMEM((CH, D), jnp.float32),
                       pltpu.VMEM((CH, D), jnp.float32),
                       pltpu.VMEM((1, tch), jnp.int32),
                       pltpu.VMEM((tch, D), jnp.float32),
                       pltpu.SemaphoreType.DMA,
                       pltpu.SemaphoreType.DMA,
                       pltpu.SemaphoreType.DMA,
                       pltpu.SemaphoreType.DMA],
    )
    def sk(dst_hbm, den_hbm, num_hbm, init_hbm, out_hbm,
           acc, iA, iB, bA, bB, iT, bT, dsA, dsB, ssA, ssB):
        c = lax.axis_index("c")
        s = lax.axis_index("s")
        base = s * epw
        nsl = pl.ds(s * NPS, NPS)
        pltpu.sync_copy(init_hbm.at[c, nsl], acc.at[nsl])
        plsc.subcore_barrier()

        def scan_rows(rows_hbm):
            def chunk2(offA, offB):
                pltpu.sync_copy(dst_hbm.at[pl.ds(e_lo + offA, CH)], iA.at[0])
                pltpu.sync_copy(dst_hbm.at[pl.ds(e_lo + offB, CH)], iB.at[0])
                cA = pltpu.async_copy(rows_hbm.at[pl.ds(offA, CH)], bA, dsA)
                cB = pltpu.async_copy(rows_hbm.at[pl.ds(offB, CH)], bB, dsB)
                cA.wait()
                sctA = pltpu.async_copy(bA, acc.at[iA.at[0]], ssA, add=True)
                cB.wait()
                sctB = pltpu.async_copy(bB, acc.at[iB.at[0]], ssB, add=True)
                sctA.wait()
                sctB.wait()

            @pl.loop(0, nch // 2)
            def _(j):
                offA = base + (2 * j) * CH
                chunk2(offA, offA + CH)

            if nch % 2:
                offO = base + (nch - 1) * CH
                pltpu.sync_copy(dst_hbm.at[pl.ds(e_lo + offO, CH)], iA.at[0])
                pltpu.sync_copy(rows_hbm.at[pl.ds(offO, CH)], bA)
                pltpu.sync_copy(bA, acc.at[iA.at[0]], add=True)
            if tch:
                offT = base + nch * CH
                pltpu.sync_copy(dst_hbm.at[pl.ds(e_lo + offT, tch)], iT.at[0])
                pltpu.sync_copy(rows_hbm.at[pl.ds(offT, tch)], bT)
                pltpu.sync_copy(bT, acc.at[iT.at[0]], add=True)

        @pl.when(c == 0)
        def _():
            scan_rows(den_hbm)

        @pl.when(c == 1)
        def _():
            scan_rows(num_hbm)

        plsc.subcore_barrier()
        pltpu.sync_copy(acc.at[nsl], out_hbm.at[c, nsl])

    return sk(dst, rows_den, rows_num, init)


def _bf16_bits(x):
    b = lax.bitcast_convert_type(x.astype(jnp.bfloat16), jnp.uint16)
    return b.astype(jnp.uint32)


def _qkv_body(h_ref, wq_ref, wk_ref, wv_ref, q_ref, kv_ref):
    D2 = D // 2
    h = h_ref[...]
    q_ref[...] = jnp.dot(h, wq_ref[...], preferred_element_type=jnp.float32)
    k = jnp.dot(h, wk_ref[...], preferred_element_type=jnp.float32)
    v = jnp.dot(h, wv_ref[...], preferred_element_type=jnp.float32)
    x = jnp.concatenate([k[:, :D2], v[:, :D2]], axis=1)
    y = jnp.concatenate([k[:, D2:], v[:, D2:]], axis=1)
    word = _bf16_bits(x) | (_bf16_bits(y) << 16)
    kv_ref[...] = lax.bitcast_convert_type(word, jnp.float32)


def _qkv(h, WQ, WK, WV):
    return pl.pallas_call(
        _qkv_body,
        out_shape=[jax.ShapeDtypeStruct((N, D), jnp.float32),
                   jax.ShapeDtypeStruct((N, D), jnp.float32)],
    )(h, WQ, WK, WV)


def _epass1_body(e_ref, kv_ref, qd_ref, we_ref, woe_ref, boe_ref,
                 ex_ref, wex_ref, z_ref, st_ref):
    i = pl.program_id(0)
    D2 = D // 2
    e = e_ref[...]
    pe = jnp.dot(e.astype(jnp.bfloat16), we_ref[...].astype(jnp.bfloat16),
                 preferred_element_type=jnp.float32)
    w = lax.bitcast_convert_type(kv_ref[...], jnp.uint32)
    lo = lax.bitcast_convert_type(w << 16, jnp.float32)
    hi = lax.bitcast_convert_type(w & jnp.uint32(0xFFFF0000), jnp.float32)
    qd = qd_ref[...]
    prod_lo = (lo[:, :D2] * qd[:, :D2]).astype(jnp.bfloat16)
    prod_hi = (hi[:, :D2] * qd[:, D2:]).astype(jnp.bfloat16)
    r2 = lax.broadcasted_iota(jnp.int32, (D2, D), 0) // DH
    cc = lax.broadcasted_iota(jnp.int32, (D2, D), 1) // DH
    Mlo = jnp.where(r2 == cc, 1.0, 0.0).astype(jnp.bfloat16)
    Mhi = jnp.where(r2 + (D2 // DH) == cc, 1.0, 0.0).astype(jnp.bfloat16)
    s0 = (jnp.dot(prod_lo, Mlo, preferred_element_type=jnp.float32)
          + jnp.dot(prod_hi, Mhi, preferred_element_type=jnp.float32)) * 0.25
    e_att = s0 + pe
    ex = jnp.exp(e_att)
    ex_ref[...] = ex
    wex_ref[...] = jnp.concatenate(
        [lo[:, D2:] * ex[:, :D2], hi[:, D2:] * ex[:, D2:]], axis=1)
    z = e + jnp.dot(e_att.astype(jnp.bfloat16),
                    woe_ref[...].astype(jnp.bfloat16),
                    preferred_element_type=jnp.float32) + boe_ref[...]
    z_ref[...] = z.astype(jnp.bfloat16)
    part = jnp.concatenate(
        [jnp.sum(z, axis=0)[None, :], jnp.sum(z * z, axis=0)[None, :],
         jnp.zeros((6, D), jnp.float32)], axis=0)

    @pl.when(i == 0)
    def _():
        st_ref[...] = part

    @pl.when(i > 0)
    def _():
        st_ref[...] = st_ref[...] + part


def _epass1(e, kvb, qdst, WE, WOe, bOe, e_cnt, blk_off):
    return pl.pallas_call(
        _epass1_body,
        grid=(e_cnt // RB,),
        in_specs=[pl.BlockSpec((RB, D), lambda i: (i + blk_off, 0)),
                  pl.BlockSpec((RB, D), lambda i: (i, 0)),
                  pl.BlockSpec((RB, D), lambda i: (i, 0)),
                  pl.BlockSpec((D, D), lambda i: (0, 0)),
                  pl.BlockSpec((D, D), lambda i: (0, 0)),
                  pl.BlockSpec((1, D), lambda i: (0, 0))],
        out_specs=[pl.BlockSpec((RB, D), lambda i: (i, 0)),
                   pl.BlockSpec((RB, D), lambda i: (i, 0)),
                   pl.BlockSpec((RB, D), lambda i: (i, 0)),
                   pl.BlockSpec((8, D), lambda i: (0, 0))],
        out_shape=[jax.ShapeDtypeStruct((e_cnt, D), jnp.float32),
                   jax.ShapeDtypeStruct((e_cnt, D), jnp.float32),
                   jax.ShapeDtypeStruct((e_cnt, D), jnp.bfloat16),
                   jax.ShapeDtypeStruct((8, D), jnp.float32)],
    )(e, kvb, qdst, WE, WOe, bOe.reshape(1, D))


def _epass2_body(z_ref, sta_ref, stb_ref, g1_ref, b1_ref, w1_ref, bf1_ref,
                 w2_ref, bf2_ref, f_ref, st2_ref):
    i = pl.program_id(0)
    st = sta_ref[...] + stb_ref[...]
    mu = st[0:1, :] / float(E)
    var = st[1:2, :] / float(E) - mu * mu
    inv = g1_ref[...] / jnp.sqrt(var + 1e-5)
    u = (z_ref[...].astype(jnp.float32) - mu) * inv + b1_ref[...]
    hid = jnp.maximum(
        jnp.dot(u.astype(jnp.bfloat16), w1_ref[...].astype(jnp.bfloat16),
                preferred_element_type=jnp.float32)
        + bf1_ref[...], 0.0)
    f = u + jnp.dot(hid.astype(jnp.bfloat16),
                    w2_ref[...].astype(jnp.bfloat16),
                    preferred_element_type=jnp.float32) + bf2_ref[...]
    f_ref[...] = f.astype(jnp.bfloat16)
    part = jnp.concatenate(
        [jnp.sum(f, axis=0)[None, :], jnp.sum(f * f, axis=0)[None, :],
         jnp.zeros((6, D), jnp.float32)], axis=0)

    @pl.when(i == 0)
    def _():
        st2_ref[...] = part

    @pl.when(i > 0)
    def _():
        st2_ref[...] = st2_ref[...] + part


def _epass2(z, st1a, st1b, g1e, b1e, W1e, bF1e, W2e, bF2e, e_cnt):
    return pl.pallas_call(
        _epass2_body,
        grid=(e_cnt // RB,),
        in_specs=[pl.BlockSpec((RB, D), lambda i: (i, 0)),
                  pl.BlockSpec((8, D), lambda i: (0, 0)),
                  pl.BlockSpec((8, D), lambda i: (0, 0)),
                  pl.BlockSpec((1, D), lambda i: (0, 0)),
                  pl.BlockSpec((1, D), lambda i: (0, 0)),
                  pl.BlockSpec((D, 2 * D), lambda i: (0, 0)),
                  pl.BlockSpec((1, 2 * D), lambda i: (0, 0)),
                  pl.BlockSpec((2 * D, D), lambda i: (0, 0)),
                  pl.BlockSpec((1, D), lambda i: (0, 0))],
        out_specs=[pl.BlockSpec((RB, D), lambda i: (i, 0)),
                   pl.BlockSpec((8, D), lambda i: (0, 0))],
        out_shape=[jax.ShapeDtypeStruct((e_cnt, D), jnp.bfloat16),
                   jax.ShapeDtypeStruct((8, D), jnp.float32)],
    )(z, st1a, st1b, g1e.reshape(1, D), b1e.reshape(1, D), W1e,
      bF1e.reshape(1, 2 * D), W2e, bF2e.reshape(1, D))


def _epass3_body(f1_ref, f2_ref, sta_ref, stb_ref, g2_ref, b2_ref, o_ref):
    i = pl.program_id(0)
    half = EH // RB
    st = sta_ref[...] + stb_ref[...]
    mu = st[0:1, :] / float(E)
    var = st[1:2, :] / float(E) - mu * mu
    inv = g2_ref[...] / jnp.sqrt(var + 1e-5)

    @pl.when(i < half)
    def _():
        o_ref[...] = (f1_ref[...].astype(jnp.float32) - mu) * inv + b2_ref[...]

    @pl.when(i >= half)
    def _():
        o_ref[...] = (f2_ref[...].astype(jnp.float32) - mu) * inv + b2_ref[...]


def _epass3(f1, f2, st2a, st2b, g2e, b2e):
    half = EH // RB
    return pl.pallas_call(
        _epass3_body,
        grid=(E // RB,),
        in_specs=[pl.BlockSpec((RB, D),
                               lambda i: (jnp.minimum(i, half - 1), 0)),
                  pl.BlockSpec((RB, D),
                               lambda i: (jnp.maximum(i - half, 0), 0)),
                  pl.BlockSpec((8, D), lambda i: (0, 0)),
                  pl.BlockSpec((8, D), lambda i: (0, 0)),
                  pl.BlockSpec((1, D), lambda i: (0, 0)),
                  pl.BlockSpec((1, D), lambda i: (0, 0))],
        out_specs=pl.BlockSpec((RB, D), lambda i: (i, 0)),
        out_shape=jax.ShapeDtypeStruct((E, D), jnp.float32),
    )(f1, f2, st2a, st2b, g2e.reshape(1, D), b2e.reshape(1, D))


def _hside_body(h_ref, sc_ref, woh_ref, boh_ref, g1_ref, b1_ref,
                w1_ref, bf1_ref, w2_ref, bf2_ref, g2_ref, b2_ref, o_ref):
    den = sc_ref[0]
    num = sc_ref[1]
    wv = jnp.where(den > 0.0, num / den, 0.0)
    h2 = h_ref[...] + jnp.dot(wv, woh_ref[...],
                              preferred_element_type=jnp.float32) + boh_ref[...]
    mu = jnp.mean(h2, axis=0, keepdims=True)
    var = jnp.mean((h2 - mu) * (h2 - mu), axis=0, keepdims=True)
    h2 = (h2 - mu) / jnp.sqrt(var + 1e-5) * g1_ref[...] + b1_ref[...]
    hid = jnp.maximum(
        jnp.dot(h2, w1_ref[...], preferred_element_type=jnp.float32)
        + bf1_ref[...], 0.0)
    h3 = h2 + jnp.dot(hid, w2_ref[...],
                      preferred_element_type=jnp.float32) + bf2_ref[...]
    mu2 = jnp.mean(h3, axis=0, keepdims=True)
    var2 = jnp.mean((h3 - mu2) * (h3 - mu2), axis=0, keepdims=True)
    o_ref[...] = (h3 - mu2) / jnp.sqrt(var2 + 1e-5) * g2_ref[...] + b2_ref[...]


def _hside(h, sc2, WOh, bOh, g1h, b1h, W1h, bF1h, W2h, bF2h,
           g2h, b2h):
    return pl.pallas_call(
        _hside_body,
        out_shape=jax.ShapeDtypeStruct((N, D), jnp.float32),
    )(h, sc2, WOh, bOh.reshape(1, D), g1h.reshape(1, D),
      b1h.reshape(1, D), W1h, bF1h.reshape(1, 2 * D), W2h,
      bF2h.reshape(1, D), g2h.reshape(1, D), b2h.reshape(1, D))


def kernel(h, e, edge_index, WQ, WK, WV, WE, WOh, bOh, WOe, bOe,
           g1h, b1h, g1e, b1e, W1h, bF1h, W2h, bF2h,
           W1e, bF1e, W2e, bF2e, g2h, b2h, g2e, b2e):
    src = edge_index[0]
    dst = edge_index[1]
    Q, KVp = _qkv(h, WQ, WK, WV)

    kv1, qd1 = _sc_gather_kvq(src, dst, KVp, Q, EH, 0)
    kv2, qd2 = _sc_gather_kvq(src, dst, KVp, Q, EH, EH)
    ex1, wex1, z1, st1a = _epass1(e, kv1, qd1, WE, WOe, bOe, EH, 0)
    ex2, wex2, z2, st1b = _epass1(e, kv2, qd2, WE, WOe, bOe, EH, EH // RB)

    zeros_nd = jnp.zeros((NC, NP, D), jnp.float32)
    sc2 = _sc_scatter2(dst, ex2, wex2,
                       _sc_scatter2(dst, ex1, wex1, zeros_nd, EH, 0),
                       EH, EH)[:, :N]
    h3 = _hside(h, sc2, WOh, bOh, g1h, b1h, W1h, bF1h, W2h, bF2h,
                g2h, b2h)
    f1, st2a = _epass2(z1, st1a, st1b, g1e, b1e, W1e, bF1e, W2e, bF2e, EH)
    f2, st2b = _epass2(z2, st1a, st1b, g1e, b1e, W1e, bF1e, W2e, bF2e, EH)
    e3 = _epass3(f1, f2, st2a, st2b, g2e, b2e)
    return h3, e3


# speedup vs baseline: 1.1960x; 1.0057x over previous
import functools

import jax
import jax.numpy as jnp
from jax import lax
from jax.experimental import pallas as pl
from jax.experimental.pallas import tpu as pltpu
from jax.experimental.pallas import tpu_sc as plsc

N = 10000
E = 320000
D = 128
H = 8
DH = 16

NC = 2
NS = 16
NW = NC * NS
EPW = E // NW
CH = 128
NCH = EPW // CH
TCH = EPW - NCH * CH
NP = 10112
NPS = NP // NS

EH = E // 2
RB = 3200


def _vmesh():
    return plsc.VectorSubcoreMesh(core_axis_name="c", subcore_axis_name="s")


def _sc_gather_kvq(src, dst, KV, Q, e_cnt, e_lo):
    epw = e_cnt // NW
    nch = epw // CH
    tch = epw - nch * CH
    @functools.partial(
        pl.kernel,
        mesh=_vmesh(),
        out_type=[jax.ShapeDtypeStruct((e_cnt, D), jnp.float32)] * 2,
        scratch_types=[pltpu.VMEM((CH,), jnp.int32),
                       pltpu.VMEM((CH,), jnp.int32),
                       pltpu.VMEM((CH,), jnp.int32),
                       pltpu.VMEM((CH,), jnp.int32),
                       pltpu.VMEM((CH, D), jnp.float32),
                       pltpu.VMEM((CH, D), jnp.float32),
                       pltpu.VMEM((CH, D), jnp.float32),
                       pltpu.VMEM((CH, D), jnp.float32),
                       pltpu.VMEM((tch, D), jnp.float32),
                       pltpu.VMEM((tch, D), jnp.float32),
                       pltpu.SemaphoreType.DMA,
                       pltpu.SemaphoreType.DMA],
    )
    def gk(src_hbm, dst_hbm, kv_hbm, q_hbm, kv_out, qd_out,
           sA, dA, sB, dB, kvA, qA, kvB, qB,
           kvT, qT, semA, semB):
        wid = lax.axis_index("c") * NS + lax.axis_index("s")
        base = wid * epw

        def run(off, n, si, di, kvb, qb, sem):
            pltpu.sync_copy(src_hbm.at[pl.ds(e_lo + off, n)],
                            si.at[pl.ds(0, n)])
            pltpu.sync_copy(dst_hbm.at[pl.ds(e_lo + off, n)],
                            di.at[pl.ds(0, n)])
            ckv = pltpu.async_copy(kv_hbm.at[si.at[pl.ds(0, n)]], kvb, sem)
            cq = pltpu.async_copy(q_hbm.at[di.at[pl.ds(0, n)]], qb, sem)
            return (ckv, cq)

        def fin(off, n, kvb, qb, cps):
            for cp in cps:
                cp.wait()
            pltpu.sync_copy(kvb, kv_out.at[pl.ds(off, n)])
            pltpu.sync_copy(qb, qd_out.at[pl.ds(off, n)])

        @pl.loop(0, nch // 2)
        def _(j):
            offA = base + (2 * j) * CH
            offB = offA + CH
            cA = run(offA, CH, sA, dA, kvA, qA, semA)
            cB = run(offB, CH, sB, dB, kvB, qB, semB)
            fin(offA, CH, kvA, qA, cA)
            fin(offB, CH, kvB, qB, cB)

        if nch % 2:
            offO = base + (nch - 1) * CH
            cO = run(offO, CH, sA, dA, kvA, qA, semA)
            fin(offO, CH, kvA, qA, cO)
        if tch:
            offT = base + nch * CH
            cT = run(offT, tch, sB, dB, kvT, qT, semB)
            fin(offT, tch, kvT, qT, cT)

    return gk(src, dst, KV, Q)


def _sc_scatter2(dst, rows_den, rows_num, init, e_cnt, e_lo):
    epw = e_cnt // NS
    nch = epw // CH
    tch = epw - nch * CH
    @functools.partial(
        pl.kernel,
        mesh=_vmesh(),
        out_type=jax.ShapeDtypeStruct((NC, NP, D), jnp.float32),
        scratch_types=[pltpu.VMEM_SHARED((NP, D), jnp.float32),
                       pltpu.VMEM((1, CH), jnp.int32),
                       pltpu.VMEM((1, CH), jnp.int32),
                       pltpu.VMEM((CH, D), jnp.float32),
                       pltpu.VMEM((CH, D), jnp.float32),
                       pltpu.VMEM((1, tch), jnp.int32),
                       pltpu.VMEM((tch, D), jnp.float32),
                       pltpu.SemaphoreType.DMA,
                       pltpu.SemaphoreType.DMA,
                       pltpu.SemaphoreType.DMA,
                       pltpu.SemaphoreType.DMA],
    )
    def sk(dst_hbm, den_hbm, num_hbm, *rest):
        if init is None:
            (out_hbm, acc, iA, iB, bA, bB, iT, bT, dsA, dsB, ssA, ssB) = rest
        else:
            (init_hbm, out_hbm, acc, iA, iB, bA, bB, iT, bT,
             dsA, dsB, ssA, ssB) = rest
        c = lax.axis_index("c")
        s = lax.axis_index("s")
        base = s * epw
        nsl = pl.ds(s * NPS, NPS)
        if init is None:
            @pl.loop(0, CH)
            def _(r):
                for cc in range(D // 16):
                    bA[r, pl.ds(cc * 16, 16)] = jnp.zeros((16,), jnp.float32)
            for k in range(NPS // CH):
                pltpu.sync_copy(bA, acc.at[pl.ds(s * NPS + k * CH, CH)])
            rem = NPS - (NPS // CH) * CH
            if rem:
                pltpu.sync_copy(
                    bA.at[pl.ds(0, rem)],
                    acc.at[pl.ds(s * NPS + (NPS // CH) * CH, rem)])
        else:
            pltpu.sync_copy(init_hbm.at[c, nsl], acc.at[nsl])
        plsc.subcore_barrier()

        def scan_rows(rows_hbm):
            def chunk2(offA, offB):
                pltpu.sync_copy(dst_hbm.at[pl.ds(e_lo + offA, CH)], iA.at[0])
                pltpu.sync_copy(dst_hbm.at[pl.ds(e_lo + offB, CH)], iB.at[0])
                cA = pltpu.async_copy(rows_hbm.at[pl.ds(offA, CH)], bA, dsA)
                cB = pltpu.async_copy(rows_hbm.at[pl.ds(offB, CH)], bB, dsB)
                cA.wait()
                sctA = pltpu.async_copy(bA, acc.at[iA.at[0]], ssA, add=True)
                cB.wait()
                sctB = pltpu.async_copy(bB, acc.at[iB.at[0]], ssB, add=True)
                sctA.wait()
                sctB.wait()

            @pl.loop(0, nch // 2)
            def _(j):
                offA = base + (2 * j) * CH
                chunk2(offA, offA + CH)

            if nch % 2:
                offO = base + (nch - 1) * CH
                pltpu.sync_copy(dst_hbm.at[pl.ds(e_lo + offO, CH)], iA.at[0])
                pltpu.sync_copy(rows_hbm.at[pl.ds(offO, CH)], bA)
                pltpu.sync_copy(bA, acc.at[iA.at[0]], add=True)
            if tch:
                offT = base + nch * CH
                pltpu.sync_copy(dst_hbm.at[pl.ds(e_lo + offT, tch)], iT.at[0])
                pltpu.sync_copy(rows_hbm.at[pl.ds(offT, tch)], bT)
                pltpu.sync_copy(bT, acc.at[iT.at[0]], add=True)

        @pl.when(c == 0)
        def _():
            scan_rows(den_hbm)

        @pl.when(c == 1)
        def _():
            scan_rows(num_hbm)

        plsc.subcore_barrier()
        pltpu.sync_copy(acc.at[nsl], out_hbm.at[c, nsl])

    if init is None:
        return sk(dst, rows_den, rows_num)
    return sk(dst, rows_den, rows_num, init)


def _bf16_bits(x):
    b = lax.bitcast_convert_type(x.astype(jnp.bfloat16), jnp.uint16)
    return b.astype(jnp.uint32)


def _qkv_body(h_ref, wq_ref, wk_ref, wv_ref, q_ref, kv_ref):
    D2 = D // 2
    h = h_ref[...]
    q_ref[...] = jnp.dot(h, wq_ref[...], preferred_element_type=jnp.float32)
    k = jnp.dot(h, wk_ref[...], preferred_element_type=jnp.float32)
    v = jnp.dot(h, wv_ref[...], preferred_element_type=jnp.float32)
    x = jnp.concatenate([k[:, :D2], v[:, :D2]], axis=1)
    y = jnp.concatenate([k[:, D2:], v[:, D2:]], axis=1)
    word = _bf16_bits(x) | (_bf16_bits(y) << 16)
    kv_ref[...] = lax.bitcast_convert_type(word, jnp.float32)


def _qkv(h, WQ, WK, WV):
    return pl.pallas_call(
        _qkv_body,
        out_shape=[jax.ShapeDtypeStruct((N, D), jnp.float32),
                   jax.ShapeDtypeStruct((N, D), jnp.float32)],
    )(h, WQ, WK, WV)


def _epass1_body(e_ref, kv_ref, qd_ref, we_ref, woe_ref, boe_ref,
                 ex_ref, wex_ref, z_ref, st_ref):
    i = pl.program_id(0)
    D2 = D // 2
    e = e_ref[...]
    pe = jnp.dot(e.astype(jnp.bfloat16), we_ref[...].astype(jnp.bfloat16),
                 preferred_element_type=jnp.float32)
    w = lax.bitcast_convert_type(kv_ref[...], jnp.uint32)
    lo = lax.bitcast_convert_type(w << 16, jnp.float32)
    hi = lax.bitcast_convert_type(w & jnp.uint32(0xFFFF0000), jnp.float32)
    qd = qd_ref[...]
    prod_lo = (lo[:, :D2] * qd[:, :D2]).astype(jnp.bfloat16)
    prod_hi = (hi[:, :D2] * qd[:, D2:]).astype(jnp.bfloat16)
    r2 = lax.broadcasted_iota(jnp.int32, (D2, D), 0) // DH
    cc = lax.broadcasted_iota(jnp.int32, (D2, D), 1) // DH
    Mlo = jnp.where(r2 == cc, 1.0, 0.0).astype(jnp.bfloat16)
    Mhi = jnp.where(r2 + (D2 // DH) == cc, 1.0, 0.0).astype(jnp.bfloat16)
    s0 = (jnp.dot(prod_lo, Mlo, preferred_element_type=jnp.float32)
          + jnp.dot(prod_hi, Mhi, preferred_element_type=jnp.float32)) * 0.25
    e_att = s0 + pe
    ex = jnp.exp(e_att)
    ex_ref[...] = ex
    wex_ref[...] = jnp.concatenate(
        [lo[:, D2:] * ex[:, :D2], hi[:, D2:] * ex[:, D2:]], axis=1)
    z = e + jnp.dot(e_att.astype(jnp.bfloat16),
                    woe_ref[...].astype(jnp.bfloat16),
                    preferred_element_type=jnp.float32) + boe_ref[...]
    z_ref[...] = z.astype(jnp.bfloat16)
    part = jnp.concatenate(
        [jnp.sum(z, axis=0)[None, :], jnp.sum(z * z, axis=0)[None, :],
         jnp.zeros((6, D), jnp.float32)], axis=0)

    @pl.when(i == 0)
    def _():
        st_ref[...] = part

    @pl.when(i > 0)
    def _():
        st_ref[...] = st_ref[...] + part


def _epass1(e, kvb, qdst, WE, WOe, bOe, e_cnt, blk_off):
    return pl.pallas_call(
        _epass1_body,
        grid=(e_cnt // RB,),
        in_specs=[pl.BlockSpec((RB, D), lambda i: (i + blk_off, 0)),
                  pl.BlockSpec((RB, D), lambda i: (i, 0)),
                  pl.BlockSpec((RB, D), lambda i: (i, 0)),
                  pl.BlockSpec((D, D), lambda i: (0, 0)),
                  pl.BlockSpec((D, D), lambda i: (0, 0)),
                  pl.BlockSpec((1, D), lambda i: (0, 0))],
        out_specs=[pl.BlockSpec((RB, D), lambda i: (i, 0)),
                   pl.BlockSpec((RB, D), lambda i: (i, 0)),
                   pl.BlockSpec((RB, D), lambda i: (i, 0)),
                   pl.BlockSpec((8, D), lambda i: (0, 0))],
        out_shape=[jax.ShapeDtypeStruct((e_cnt, D), jnp.float32),
                   jax.ShapeDtypeStruct((e_cnt, D), jnp.float32),
                   jax.ShapeDtypeStruct((e_cnt, D), jnp.bfloat16),
                   jax.ShapeDtypeStruct((8, D), jnp.float32)],
    )(e, kvb, qdst, WE, WOe, bOe.reshape(1, D))


def _epass2_body(z_ref, sta_ref, stb_ref, g1_ref, b1_ref, w1_ref, bf1_ref,
                 w2_ref, bf2_ref, f_ref, st2_ref):
    i = pl.program_id(0)
    st = sta_ref[...] + stb_ref[...]
    mu = st[0:1, :] / float(E)
    var = st[1:2, :] / float(E) - mu * mu
    inv = g1_ref[...] / jnp.sqrt(var + 1e-5)
    u = (z_ref[...].astype(jnp.float32) - mu) * inv + b1_ref[...]
    hid = jnp.maximum(
        jnp.dot(u.astype(jnp.bfloat16), w1_ref[...].astype(jnp.bfloat16),
                preferred_element_type=jnp.float32)
        + bf1_ref[...], 0.0)
    f = u + jnp.dot(hid.astype(jnp.bfloat16),
                    w2_ref[...].astype(jnp.bfloat16),
                    preferred_element_type=jnp.float32) + bf2_ref[...]
    f_ref[...] = f.astype(jnp.bfloat16)
    part = jnp.concatenate(
        [jnp.sum(f, axis=0)[None, :], jnp.sum(f * f, axis=0)[None, :],
         jnp.zeros((6, D), jnp.float32)], axis=0)

    @pl.when(i == 0)
    def _():
        st2_ref[...] = part

    @pl.when(i > 0)
    def _():
        st2_ref[...] = st2_ref[...] + part


def _epass2(z, st1a, st1b, g1e, b1e, W1e, bF1e, W2e, bF2e, e_cnt):
    return pl.pallas_call(
        _epass2_body,
        grid=(e_cnt // RB,),
        in_specs=[pl.BlockSpec((RB, D), lambda i: (i, 0)),
                  pl.BlockSpec((8, D), lambda i: (0, 0)),
                  pl.BlockSpec((8, D), lambda i: (0, 0)),
                  pl.BlockSpec((1, D), lambda i: (0, 0)),
                  pl.BlockSpec((1, D), lambda i: (0, 0)),
                  pl.BlockSpec((D, 2 * D), lambda i: (0, 0)),
                  pl.BlockSpec((1, 2 * D), lambda i: (0, 0)),
                  pl.BlockSpec((2 * D, D), lambda i: (0, 0)),
                  pl.BlockSpec((1, D), lambda i: (0, 0))],
        out_specs=[pl.BlockSpec((RB, D), lambda i: (i, 0)),
                   pl.BlockSpec((8, D), lambda i: (0, 0))],
        out_shape=[jax.ShapeDtypeStruct((e_cnt, D), jnp.bfloat16),
                   jax.ShapeDtypeStruct((8, D), jnp.float32)],
    )(z, st1a, st1b, g1e.reshape(1, D), b1e.reshape(1, D), W1e,
      bF1e.reshape(1, 2 * D), W2e, bF2e.reshape(1, D))


def _epass3_body(f1_ref, f2_ref, sta_ref, stb_ref, g2_ref, b2_ref, o_ref):
    i = pl.program_id(0)
    half = EH // RB
    st = sta_ref[...] + stb_ref[...]
    mu = st[0:1, :] / float(E)
    var = st[1:2, :] / float(E) - mu * mu
    inv = g2_ref[...] / jnp.sqrt(var + 1e-5)

    @pl.when(i < half)
    def _():
        o_ref[...] = (f1_ref[...].astype(jnp.float32) - mu) * inv + b2_ref[...]

    @pl.when(i >= half)
    def _():
        o_ref[...] = (f2_ref[...].astype(jnp.float32) - mu) * inv + b2_ref[...]


def _epass3(f1, f2, st2a, st2b, g2e, b2e):
    half = EH // RB
    return pl.pallas_call(
        _epass3_body,
        grid=(E // RB,),
        in_specs=[pl.BlockSpec((RB, D),
                               lambda i: (jnp.minimum(i, half - 1), 0)),
                  pl.BlockSpec((RB, D),
                               lambda i: (jnp.maximum(i - half, 0), 0)),
                  pl.BlockSpec((8, D), lambda i: (0, 0)),
                  pl.BlockSpec((8, D), lambda i: (0, 0)),
                  pl.BlockSpec((1, D), lambda i: (0, 0)),
                  pl.BlockSpec((1, D), lambda i: (0, 0))],
        out_specs=pl.BlockSpec((RB, D), lambda i: (i, 0)),
        out_shape=jax.ShapeDtypeStruct((E, D), jnp.float32),
    )(f1, f2, st2a, st2b, g2e.reshape(1, D), b2e.reshape(1, D))


def _hside_body(h_ref, sc_ref, woh_ref, boh_ref, g1_ref, b1_ref,
                w1_ref, bf1_ref, w2_ref, bf2_ref, g2_ref, b2_ref, o_ref):
    den = sc_ref[0]
    num = sc_ref[1]
    wv = jnp.where(den > 0.0, num / den, 0.0)
    h2 = h_ref[...] + jnp.dot(wv, woh_ref[...],
                              preferred_element_type=jnp.float32) + boh_ref[...]
    mu = jnp.mean(h2, axis=0, keepdims=True)
    var = jnp.mean((h2 - mu) * (h2 - mu), axis=0, keepdims=True)
    h2 = (h2 - mu) / jnp.sqrt(var + 1e-5) * g1_ref[...] + b1_ref[...]
    hid = jnp.maximum(
        jnp.dot(h2, w1_ref[...], preferred_element_type=jnp.float32)
        + bf1_ref[...], 0.0)
    h3 = h2 + jnp.dot(hid, w2_ref[...],
                      preferred_element_type=jnp.float32) + bf2_ref[...]
    mu2 = jnp.mean(h3, axis=0, keepdims=True)
    var2 = jnp.mean((h3 - mu2) * (h3 - mu2), axis=0, keepdims=True)
    o_ref[...] = (h3 - mu2) / jnp.sqrt(var2 + 1e-5) * g2_ref[...] + b2_ref[...]


def _hside(h, sc2, WOh, bOh, g1h, b1h, W1h, bF1h, W2h, bF2h,
           g2h, b2h):
    return pl.pallas_call(
        _hside_body,
        out_shape=jax.ShapeDtypeStruct((N, D), jnp.float32),
    )(h, sc2, WOh, bOh.reshape(1, D), g1h.reshape(1, D),
      b1h.reshape(1, D), W1h, bF1h.reshape(1, 2 * D), W2h,
      bF2h.reshape(1, D), g2h.reshape(1, D), b2h.reshape(1, D))


def kernel(h, e, edge_index, WQ, WK, WV, WE, WOh, bOh, WOe, bOe,
           g1h, b1h, g1e, b1e, W1h, bF1h, W2h, bF2h,
           W1e, bF1e, W2e, bF2e, g2h, b2h, g2e, b2e):
    src = edge_index[0]
    dst = edge_index[1]
    Q, KVp = _qkv(h, WQ, WK, WV)

    kv1, qd1 = _sc_gather_kvq(src, dst, KVp, Q, EH, 0)
    kv2, qd2 = _sc_gather_kvq(src, dst, KVp, Q, EH, EH)
    ex1, wex1, z1, st1a = _epass1(e, kv1, qd1, WE, WOe, bOe, EH, 0)
    ex2, wex2, z2, st1b = _epass1(e, kv2, qd2, WE, WOe, bOe, EH, EH // RB)

    f1, st2a = _epass2(z1, st1a, st1b, g1e, b1e, W1e, bF1e, W2e, bF2e, EH)
    f2, st2b = _epass2(z2, st1a, st1b, g1e, b1e, W1e, bF1e, W2e, bF2e, EH)
    sc2 = _sc_scatter2(dst, ex2, wex2,
                       _sc_scatter2(dst, ex1, wex1, None, EH, 0),
                       EH, EH)[:, :N]
    h3 = _hside(h, sc2, WOh, bOh, g1h, b1h, W1h, bF1h, W2h, bF2h,
                g2h, b2h)
    e3 = _epass3(f1, f2, st2a, st2b, g2e, b2e)
    return h3, e3
